# baseline (speedup 1.0000x reference)
import dataclasses
import functools

import jax
import jax.numpy as jnp
from jax import lax
from jax.experimental import pallas as pl
from jax.experimental.pallas import tpu as pltpu
from jax.experimental.pallas import tpu_sc as plsc

GAMMA_ = 12.0
N_ = 10000
E_ = 320000
D_ = 128
NC_ = 2
NS_ = 16
L_ = 16
NW_ = NC_ * NS_
EPT_ = E_ // NW_
B_ = 40
NBLK_ = EPT_ // B_
RPT_ = 624
REM_ = N_ - NS_ * RPT_
NRING_ = 4


def _edge_block_compute(gv, relv, msgv):
    himask = jnp.full((L_,), -65536, jnp.int32)

    @pl.loop(0, B_, step=2)
    def _(e0):
        for e in (e0, e0 + 1):
            accs = [jnp.zeros((L_,), jnp.float32) for _ in range(4)]
            trs = []
            rs = []
            for j2 in range(D_ // (2 * L_)):
                w = relv[e, pl.ds(L_ * j2, L_)]
                rs.append(lax.bitcast_convert_type(w << 16, jnp.float32))
                rs.append(lax.bitcast_convert_type(w & himask, jnp.float32))
            for j in range(D_ // L_):
                h = gv[e, pl.ds(L_ * j, L_)]
                r = rs[j]
                t = gv[B_ + e, pl.ds(L_ * j, L_)]
                tr = h + r
                d = tr - t
                accs[j % 4] = accs[j % 4] + d * d
                trs.append(tr)
            acc = (accs[0] + accs[1]) + (accs[2] + accs[3])
            d2 = jnp.broadcast_to(jnp.sum(acc), (L_,))
            bits = lax.bitcast_convert_type(d2, jnp.int32)
            seed = jnp.full((L_,), 0x5F3759DF, jnp.int32) - (bits >> 1)
            y = lax.bitcast_convert_type(seed, jnp.float32)
            half = d2 * 0.5
            y = y * (1.5 - half * y * y)
            y = y * (1.5 - half * y * y)
            dist = d2 * y
            score = 1.0 / (1.0 + jnp.exp(dist - GAMMA_))
            for j in range(D_ // L_):
                msgv[e, pl.ds(L_ * j, L_)] = trs[j] * score


def _sc_partials(x, idxg, idxs, rel, zrows):
    mesh = plsc.VectorSubcoreMesh(core_axis_name="c", subcore_axis_name="s")
    cp = pltpu.CompilerParams()
    if "needs_layout_passes" in pltpu.CompilerParams.__dataclass_fields__:
        cp = dataclasses.replace(cp, needs_layout_passes=False)

    @functools.partial(
        pl.kernel,
        compiler_params=cp,
        out_type=jax.ShapeDtypeStruct((NC_ * N_, D_), jnp.float32),
        mesh=mesh,
        scratch_types=[
            pltpu.VMEM((NRING_, 1, 2 * B_), jnp.int32),
            pltpu.VMEM((NRING_, 1, B_), jnp.int32),
            pltpu.VMEM((2 * B_, D_), jnp.float32),
            pltpu.VMEM((2 * B_, D_), jnp.float32),
            pltpu.VMEM((B_, D_ // 2), jnp.int32),
            pltpu.VMEM((B_, D_ // 2), jnp.int32),
            pltpu.VMEM((B_, D_), jnp.float32),
            pltpu.VMEM_SHARED((N_, D_), jnp.float32),
            pltpu.SemaphoreType.DMA,
            pltpu.SemaphoreType.DMA,
            pltpu.SemaphoreType.DMA,
            pltpu.SemaphoreType.DMA,
        ],
    )
    def k(x_hbm, idxg_hbm, idxs_hbm, rel_hbm, z_hbm, out_hbm,
          ringG, ringS, gv0, gv1, relv0, relv1, msgv, hsh,
          semd0, semd1, semi, sems):
        cid = lax.axis_index("c")
        sid = lax.axis_index("s")
        wid = sid * NC_ + cid
        gv = (gv0, gv1)
        relv = (relv0, relv1)
        semd = (semd0, semd1)

        pltpu.sync_copy(z_hbm, hsh.at[pl.ds(sid * RPT_, RPT_)])

        @pl.when(sid == 0)
        def _():
            pltpu.sync_copy(z_hbm.at[pl.ds(0, REM_)],
                            hsh.at[pl.ds(NS_ * RPT_, REM_)])

        plsc.subcore_barrier()

        def issue_idx(b):
            slot = lax.rem(b, NRING_)
            pltpu.async_copy(idxg_hbm.at[wid, b], ringG.at[slot], semi)
            pltpu.async_copy(idxs_hbm.at[wid, b], ringS.at[slot], semi)

        def wait_idx():
            pltpu.make_async_copy(idxg_hbm.at[wid, 0], ringG.at[0],
                                  semi).wait()
            pltpu.make_async_copy(idxs_hbm.at[wid, 0], ringS.at[0],
                                  semi).wait()

        def issue2(b, buf):
            slot = lax.rem(b, NRING_)
            pltpu.async_copy(x_hbm.at[ringG.at[slot, 0]], gv[buf], semd[buf])
            pltpu.async_copy(rel_hbm.at[pl.ds(wid * EPT_ + b * B_, B_)],
                             relv[buf], semd[buf])

        def wait2(buf):
            pltpu.make_async_copy(x_hbm.at[pl.ds(0, 2 * B_)], gv[buf],
                                  semd[buf]).wait()
            pltpu.make_async_copy(rel_hbm.at[pl.ds(0, B_)], relv[buf],
                                  semd[buf]).wait()

        def wait_scatter():
            pltpu.make_async_copy(msgv, hsh.at[pl.ds(0, B_)], sems).wait()

        def step(b, buf):
            wait2(buf)

            @pl.when(b > 0)
            def _():
                wait_scatter()

            _edge_block_compute(gv[buf], relv[buf], msgv)
            slot = lax.rem(b, NRING_)
            pltpu.async_copy(msgv, hsh.at[ringS.at[slot, 0]], sems, add=True)

            @pl.when(b + 2 < NBLK_)
            def _():
                wait_idx()
                issue2(b + 2, buf)

            @pl.when(b + 3 < NBLK_)
            def _():
                issue_idx(b + 3)

        pltpu.sync_copy(idxg_hbm.at[wid, 0], ringG.at[0])
        pltpu.sync_copy(idxs_hbm.at[wid, 0], ringS.at[0])
        pltpu.sync_copy(idxg_hbm.at[wid, 1], ringG.at[1])
        pltpu.sync_copy(idxs_hbm.at[wid, 1], ringS.at[1])
        issue2(0, 0)
        issue2(1, 1)
        issue_idx(2)

        @pl.loop(0, NBLK_ // 2)
        def _(i):
            step(2 * i, 0)
            step(2 * i + 1, 1)

        wait_scatter()
        plsc.subcore_barrier()
        pltpu.sync_copy(
            hsh.at[pl.ds(sid * RPT_, RPT_)],
            out_hbm.at[pl.ds(cid * N_ + sid * RPT_, RPT_)],
        )

        @pl.when(sid == 0)
        def _():
            pltpu.sync_copy(
                hsh.at[pl.ds(NS_ * RPT_, REM_)],
                out_hbm.at[pl.ds(cid * N_ + NS_ * RPT_, REM_)],
            )

    return k(x, idxg, idxs, rel, zrows)


def _combine(partials):
    bn = 2000

    def add_k(p_ref, o_ref):
        o_ref[...] = p_ref[0] + p_ref[1]

    return pl.pallas_call(
        add_k,
        out_shape=jax.ShapeDtypeStruct((N_, D_), jnp.float32),
        grid=(N_ // bn,),
        in_specs=[pl.BlockSpec((2, bn, D_), lambda i: (0, i, 0))],
        out_specs=pl.BlockSpec((bn, D_), lambda i: (i, 0)),
    )(partials)


@jax.jit
def kernel(x, edge_index, edge_attr):
    ei = edge_index.astype(jnp.int32).reshape(2, NW_, NBLK_, 1, B_)
    idxg = ei.transpose(1, 2, 3, 0, 4).reshape(NW_, NBLK_, 1, 2 * B_)
    idxs = ei[1]
    rel4 = edge_attr.reshape(E_, 4, 2, L_).transpose(0, 1, 3, 2)
    rel_i32 = lax.bitcast_convert_type(
        rel4.astype(jnp.bfloat16), jnp.int32).reshape(E_, D_ // 2)
    zrows = jnp.zeros((RPT_, D_), jnp.float32)
    partials = _sc_partials(x, idxg, idxs, rel_i32, zrows)
    return _combine(partials.reshape(NC_, N_, D_))


# speedup vs baseline: 2.3116x; 2.3116x over previous
import dataclasses
import functools

import jax
import jax.numpy as jnp
from jax import lax
from jax.experimental import pallas as pl
from jax.experimental.pallas import tpu as pltpu
from jax.experimental.pallas import tpu_sc as plsc

GAMMA_ = 12.0
N_ = 10000
E_ = 320000
D_ = 128
NC_ = 2
NS_ = 16
L_ = 16
NW_ = NC_ * NS_
EPT_ = E_ // NW_
B_ = 40
NBLK_ = EPT_ // B_
RPT_ = 624
REM_ = N_ - NS_ * RPT_
NRING_ = 4


def _edge_block_compute(gv, relv, msgv):
    @pl.loop(0, B_, step=2)
    def _(e0):
        for e in (e0, e0 + 1):
            accs = [jnp.zeros((L_,), jnp.float32) for _ in range(4)]
            trs = []
            for j in range(D_ // L_):
                h = gv[e, pl.ds(L_ * j, L_)]
                r = relv[e, pl.ds(L_ * j, L_)]
                t = gv[B_ + e, pl.ds(L_ * j, L_)]
                tr = h + r
                d = tr - t
                accs[j % 4] = accs[j % 4] + d * d
                trs.append(tr)
            acc = (accs[0] + accs[1]) + (accs[2] + accs[3])
            d2 = jnp.broadcast_to(jnp.sum(acc), (L_,))
            bits = lax.bitcast_convert_type(d2, jnp.int32)
            seed = jnp.full((L_,), 0x5F3759DF, jnp.int32) - (bits >> 1)
            y = lax.bitcast_convert_type(seed, jnp.float32)
            half = d2 * 0.5
            y = y * (1.5 - half * y * y)
            y = y * (1.5 - half * y * y)
            dist = d2 * y
            score = 1.0 / (1.0 + jnp.exp(dist - GAMMA_))
            for j in range(D_ // L_):
                msgv[e, pl.ds(L_ * j, L_)] = trs[j] * score


def _sc_partials(x, idxg, idxs, rel, zrows):
    mesh = plsc.VectorSubcoreMesh(core_axis_name="c", subcore_axis_name="s")
    cp = pltpu.CompilerParams()
    if "needs_layout_passes" in pltpu.CompilerParams.__dataclass_fields__:
        cp = dataclasses.replace(cp, needs_layout_passes=False)

    @functools.partial(
        pl.kernel,
        compiler_params=cp,
        out_type=jax.ShapeDtypeStruct((NC_ * N_, D_), jnp.float32),
        mesh=mesh,
        scratch_types=[
            pltpu.VMEM((NRING_, 1, 2 * B_), jnp.int32),
            pltpu.VMEM((NRING_, 1, B_), jnp.int32),
            pltpu.VMEM((2 * B_, D_), jnp.float32),
            pltpu.VMEM((2 * B_, D_), jnp.float32),
            pltpu.VMEM((B_, D_), jnp.float32),
            pltpu.VMEM((B_, D_), jnp.float32),
            pltpu.VMEM((B_, D_), jnp.float32),
            pltpu.VMEM_SHARED((N_, D_), jnp.float32),
            pltpu.SemaphoreType.DMA,
            pltpu.SemaphoreType.DMA,
            pltpu.SemaphoreType.DMA,
            pltpu.SemaphoreType.DMA,
        ],
    )
    def k(x_hbm, idxg_hbm, idxs_hbm, rel_hbm, z_hbm, out_hbm,
          ringG, ringS, gv0, gv1, relv0, relv1, msgv, hsh,
          semd0, semd1, semi, sems):
        cid = lax.axis_index("c")
        sid = lax.axis_index("s")
        wid = sid * NC_ + cid
        gv = (gv0, gv1)
        relv = (relv0, relv1)
        semd = (semd0, semd1)

        pltpu.sync_copy(z_hbm, hsh.at[pl.ds(sid * RPT_, RPT_)])

        @pl.when(sid == 0)
        def _():
            pltpu.sync_copy(z_hbm.at[pl.ds(0, REM_)],
                            hsh.at[pl.ds(NS_ * RPT_, REM_)])

        plsc.subcore_barrier()

        def issue_idx(b):
            slot = lax.rem(b, NRING_)
            pltpu.async_copy(idxg_hbm.at[wid, b], ringG.at[slot], semi)
            pltpu.async_copy(idxs_hbm.at[wid, b], ringS.at[slot], semi)

        def wait_idx():
            pltpu.make_async_copy(idxg_hbm.at[wid, 0], ringG.at[0],
                                  semi).wait()
            pltpu.make_async_copy(idxs_hbm.at[wid, 0], ringS.at[0],
                                  semi).wait()

        def issue2(b, buf):
            slot = lax.rem(b, NRING_)
            pltpu.async_copy(x_hbm.at[ringG.at[slot, 0]], gv[buf], semd[buf])
            pltpu.async_copy(rel_hbm.at[pl.ds(wid * EPT_ + b * B_, B_)],
                             relv[buf], semd[buf])

        def wait2(buf):
            pltpu.make_async_copy(rel_hbm.at[pl.ds(0, 2 * B_)], gv[buf],
                                  semd[buf]).wait()
            pltpu.make_async_copy(rel_hbm.at[pl.ds(0, B_)], relv[buf],
                                  semd[buf]).wait()

        def wait_scatter():
            pltpu.make_async_copy(msgv, hsh.at[pl.ds(0, B_)], sems).wait()

        def step(b, buf):
            wait2(buf)

            @pl.when(b > 0)
            def _():
                wait_scatter()

            _edge_block_compute(gv[buf], relv[buf], msgv)
            slot = lax.rem(b, NRING_)
            pltpu.async_copy(msgv, hsh.at[ringS.at[slot, 0]], sems, add=True)

            @pl.when(b + 2 < NBLK_)
            def _():
                wait_idx()
                issue2(b + 2, buf)

            @pl.when(b + 3 < NBLK_)
            def _():
                issue_idx(b + 3)

        pltpu.sync_copy(idxg_hbm.at[wid, 0], ringG.at[0])
        pltpu.sync_copy(idxs_hbm.at[wid, 0], ringS.at[0])
        pltpu.sync_copy(idxg_hbm.at[wid, 1], ringG.at[1])
        pltpu.sync_copy(idxs_hbm.at[wid, 1], ringS.at[1])
        issue2(0, 0)
        issue2(1, 1)
        issue_idx(2)

        @pl.loop(0, NBLK_ // 2)
        def _(i):
            step(2 * i, 0)
            step(2 * i + 1, 1)

        wait_scatter()
        plsc.subcore_barrier()
        pltpu.sync_copy(
            hsh.at[pl.ds(sid * RPT_, RPT_)],
            out_hbm.at[pl.ds(cid * N_ + sid * RPT_, RPT_)],
        )

        @pl.when(sid == 0)
        def _():
            pltpu.sync_copy(
                hsh.at[pl.ds(NS_ * RPT_, REM_)],
                out_hbm.at[pl.ds(cid * N_ + NS_ * RPT_, REM_)],
            )

    return k(x, idxg, idxs, rel, zrows)


def _combine(partials):
    bn = 2000

    def add_k(p_ref, o_ref):
        o_ref[...] = p_ref[0] + p_ref[1]

    return pl.pallas_call(
        add_k,
        out_shape=jax.ShapeDtypeStruct((N_, D_), jnp.float32),
        grid=(N_ // bn,),
        in_specs=[pl.BlockSpec((2, bn, D_), lambda i: (0, i, 0))],
        out_specs=pl.BlockSpec((bn, D_), lambda i: (i, 0)),
    )(partials)


@jax.jit
def kernel(x, edge_index, edge_attr):
    ei = edge_index.astype(jnp.int32).reshape(2, NW_, NBLK_, 1, B_)
    idxg = ei.transpose(1, 2, 3, 0, 4).reshape(NW_, NBLK_, 1, 2 * B_)
    idxs = ei[1]
    zrows = jnp.zeros((RPT_, D_), jnp.float32)
    partials = _sc_partials(x, idxg, idxs, edge_attr, zrows)
    return _combine(partials.reshape(NC_, N_, D_))
